# baseline (speedup 1.0000x reference)
import jax
import jax.numpy as jnp
from jax import lax
from jax.experimental import pallas as pl
from jax.experimental.pallas import tpu as pltpu
from jax.experimental.pallas import tpu_sc as plsc

BATCH = 16384
DIM = 128

_NC = 2
_NS = 16
_NW = _NC * _NS
_CHUNK = 128
_SPLITS = (12288, 4096)


def _make_sc_gather_body(row0, srows):
    _NCHUNK = srows // (_NW * _CHUNK)

    def body(uidx_hbm, iidx_hbm, user_emb_hbm, item_emb_hbm,
             x_hbm,
             uidx_v, iidx_v,
             ur0, ur1, ir0, ir1,
             g0, g1, s0, s1):
        wid = lax.axis_index("s") * _NC + lax.axis_index("c")
        idx_row0 = row0 // _CHUNK + wid * _NCHUNK
        ubuf, ibuf = (ur0, ur1), (ir0, ir1)
        gsem, ssem = (g0, g1), (s0, s1)
        for c in range(_NCHUNK):
            pltpu.sync_copy(uidx_hbm.at[pl.ds(idx_row0 + c, 1)],
                            uidx_v.at[pl.ds(c, 1)])
            pltpu.sync_copy(iidx_hbm.at[pl.ds(idx_row0 + c, 1)],
                            iidx_v.at[pl.ds(c, 1)])

        def fire_gather(c, s):
            cu = pltpu.async_copy(user_emb_hbm.at[uidx_v.at[c]], ubuf[s], gsem[s])
            ci = pltpu.async_copy(item_emb_hbm.at[iidx_v.at[c]], ibuf[s], gsem[s])
            return cu, ci

        scat = [None, None]
        gath = [None, None]
        gath[0] = fire_gather(0, 0)
        for c in range(_NCHUNK):
            s = c % 2
            if c + 1 < _NCHUNK:
                if scat[1 - s] is not None:
                    scat[1 - s][0].wait()
                    scat[1 - s][1].wait()
                gath[1 - s] = fire_gather(c + 1, 1 - s)
            gath[s][0].wait()
            gath[s][1].wait()
            blk = wid * _NCHUNK + c
            su = pltpu.async_copy(
                ubuf[s], x_hbm.at[blk, slice(None), pl.ds(0, DIM)], ssem[s])
            si = pltpu.async_copy(
                ibuf[s], x_hbm.at[blk, slice(None), pl.ds(DIM, DIM)], ssem[s])
            scat[s] = (su, si)
        for s in range(2):
            if scat[s] is not None:
                scat[s][0].wait()
                scat[s][1].wait()

    return body


def _sc_gather(row0, srows, uidx, iidx, user_emb, item_emb):
    nchunk = srows // (_NW * _CHUNK)
    mesh = plsc.VectorSubcoreMesh(core_axis_name="c", subcore_axis_name="s")
    return pl.kernel(
        _make_sc_gather_body(row0, srows),
        out_type=jax.ShapeDtypeStruct((srows // _CHUNK, _CHUNK, 2 * DIM),
                                      jnp.float32),
        mesh=mesh,
        compiler_params=pltpu.CompilerParams(use_tc_tiling_on_sc=True),
        scratch_types=[
            pltpu.VMEM((nchunk, _CHUNK), jnp.int32),
            pltpu.VMEM((nchunk, _CHUNK), jnp.int32),
            pltpu.VMEM((_CHUNK, DIM), jnp.float32),
            pltpu.VMEM((_CHUNK, DIM), jnp.float32),
            pltpu.VMEM((_CHUNK, DIM), jnp.float32),
            pltpu.VMEM((_CHUNK, DIM), jnp.float32),
            pltpu.SemaphoreType.DMA,
            pltpu.SemaphoreType.DMA,
            pltpu.SemaphoreType.DMA,
            pltpu.SemaphoreType.DMA,
        ],
    )(uidx, iidx, user_emb, item_emb)


_TILE = 4096


def _dot_t(x, w):
    return lax.dot_general(x, w, (((1,), (1,)), ((), ())),
                           preferred_element_type=jnp.float32)


def _mlp_body(x_ref, w0_ref, b0_ref, w1_ref, b1_ref,
              w2_ref, b2_ref, w3_ref, b3_ref, out_ref):
    x = x_ref[...].reshape(_TILE, 2 * DIM)
    h = jnp.maximum(_dot_t(x, w0_ref[...]) + b0_ref[...], 0.0)
    h = jnp.maximum(_dot_t(h, w1_ref[...]) + b1_ref[...], 0.0)
    h = jnp.maximum(_dot_t(h, w2_ref[...]) + b2_ref[...], 0.0)
    h = jnp.dot(h, w3_ref[...], preferred_element_type=jnp.float32)
    h = jnp.maximum(h + b3_ref[...], 0.0)
    out_ref[...] = 1.0 / (1.0 + jnp.exp(-h))


def _mlp(x, w0, b0, w1, b1, w2, b2, w3, b3):
    rows = x.shape[0] * _CHUNK
    grid = (rows // _TILE,)
    full = lambda shape: pl.BlockSpec(shape, lambda i: (0, 0))
    return pl.pallas_call(
        _mlp_body,
        grid=grid,
        in_specs=[
            pl.BlockSpec((_TILE // _CHUNK, _CHUNK, 2 * DIM),
                         lambda i: (i, 0, 0)),
            full(w0.shape), full(b0.shape),
            full(w1.shape), full(b1.shape),
            full(w2.shape), full(b2.shape),
            full(w3.shape), full(b3.shape),
        ],
        out_specs=pl.BlockSpec((_TILE, 1), lambda i: (i, 0)),
        out_shape=jax.ShapeDtypeStruct((rows, 1), jnp.float32),
    )(x, w0, b0, w1, b1, w2, b2, w3, b3)


def kernel(item_index, user_index, item_emb, user_emb,
           W0, b0, W1, b1, W2, b2, W3, b3):
    uidx = user_index.astype(jnp.int32).reshape(BATCH // _CHUNK, _CHUNK)
    iidx = item_index.astype(jnp.int32).reshape(BATCH // _CHUNK, _CHUNK)

    wargs = (W0, b0.reshape(1, -1),
             W1, b1.reshape(1, -1),
             W2, b2.reshape(1, -1),
             W3.T, b3.reshape(1, -1))

    xs = []
    row0 = 0
    for srows in _SPLITS:
        xs.append(_sc_gather(row0, srows, uidx, iidx, user_emb, item_emb))
        row0 += srows
    outs = [_mlp(x, *wargs) for x in xs]
    if len(outs) == 1:
        return outs[0]
    return jnp.concatenate(outs, axis=0)


# speedup vs baseline: 1.0012x; 1.0012x over previous
import jax
import jax.numpy as jnp
from jax import lax
from jax.experimental import pallas as pl
from jax.experimental.pallas import tpu as pltpu
from jax.experimental.pallas import tpu_sc as plsc

BATCH = 16384
DIM = 128

_NC = 2
_NS = 16
_NW = _NC * _NS
_CHUNK = 128
_SPLITS = (4096, 12288)


def _make_sc_gather_body(row0, srows):
    _NCHUNK = srows // (_NW * _CHUNK)

    def body(uidx_hbm, iidx_hbm, user_emb_hbm, item_emb_hbm,
             x_hbm,
             uidx_v, iidx_v,
             ur0, ur1, ir0, ir1,
             g0, g1, s0, s1):
        wid = lax.axis_index("s") * _NC + lax.axis_index("c")
        idx_row0 = row0 // _CHUNK + wid * _NCHUNK
        ubuf, ibuf = (ur0, ur1), (ir0, ir1)
        gsem, ssem = (g0, g1), (s0, s1)
        for c in range(_NCHUNK):
            pltpu.sync_copy(uidx_hbm.at[pl.ds(idx_row0 + c, 1)],
                            uidx_v.at[pl.ds(c, 1)])
            pltpu.sync_copy(iidx_hbm.at[pl.ds(idx_row0 + c, 1)],
                            iidx_v.at[pl.ds(c, 1)])

        def fire_gather(c, s):
            cu = pltpu.async_copy(user_emb_hbm.at[uidx_v.at[c]], ubuf[s], gsem[s])
            ci = pltpu.async_copy(item_emb_hbm.at[iidx_v.at[c]], ibuf[s], gsem[s])
            return cu, ci

        scat = [None, None]
        gath = [None, None]
        gath[0] = fire_gather(0, 0)
        for c in range(_NCHUNK):
            s = c % 2
            if c + 1 < _NCHUNK:
                if scat[1 - s] is not None:
                    scat[1 - s][0].wait()
                    scat[1 - s][1].wait()
                gath[1 - s] = fire_gather(c + 1, 1 - s)
            gath[s][0].wait()
            gath[s][1].wait()
            blk = wid * _NCHUNK + c
            su = pltpu.async_copy(
                ubuf[s], x_hbm.at[blk, slice(None), pl.ds(0, DIM)], ssem[s])
            si = pltpu.async_copy(
                ibuf[s], x_hbm.at[blk, slice(None), pl.ds(DIM, DIM)], ssem[s])
            scat[s] = (su, si)
        for s in range(2):
            if scat[s] is not None:
                scat[s][0].wait()
                scat[s][1].wait()

    return body


def _sc_gather(row0, srows, uidx, iidx, user_emb, item_emb):
    nchunk = srows // (_NW * _CHUNK)
    mesh = plsc.VectorSubcoreMesh(core_axis_name="c", subcore_axis_name="s")
    return pl.kernel(
        _make_sc_gather_body(row0, srows),
        out_type=jax.ShapeDtypeStruct((srows // _CHUNK, _CHUNK, 2 * DIM),
                                      jnp.float32),
        mesh=mesh,
        compiler_params=pltpu.CompilerParams(use_tc_tiling_on_sc=True),
        scratch_types=[
            pltpu.VMEM((nchunk, _CHUNK), jnp.int32),
            pltpu.VMEM((nchunk, _CHUNK), jnp.int32),
            pltpu.VMEM((_CHUNK, DIM), jnp.float32),
            pltpu.VMEM((_CHUNK, DIM), jnp.float32),
            pltpu.VMEM((_CHUNK, DIM), jnp.float32),
            pltpu.VMEM((_CHUNK, DIM), jnp.float32),
            pltpu.SemaphoreType.DMA,
            pltpu.SemaphoreType.DMA,
            pltpu.SemaphoreType.DMA,
            pltpu.SemaphoreType.DMA,
        ],
    )(uidx, iidx, user_emb, item_emb)


_TILE = 4096


def _dot_t(x, w):
    return lax.dot_general(x, w, (((1,), (1,)), ((), ())),
                           preferred_element_type=jnp.float32)


def _mlp_body(x_ref, w0_ref, b0_ref, w1_ref, b1_ref,
              w2_ref, b2_ref, w3_ref, b3_ref, out_ref):
    x = x_ref[...].reshape(_TILE, 2 * DIM)
    h = jnp.maximum(_dot_t(x, w0_ref[...]) + b0_ref[...], 0.0)
    h = jnp.maximum(_dot_t(h, w1_ref[...]) + b1_ref[...], 0.0)
    h = jnp.maximum(_dot_t(h, w2_ref[...]) + b2_ref[...], 0.0)
    h = jnp.dot(h, w3_ref[...], preferred_element_type=jnp.float32)
    h = jnp.maximum(h + b3_ref[...], 0.0)
    out_ref[...] = 1.0 / (1.0 + jnp.exp(-h))


def _mlp(x, w0, b0, w1, b1, w2, b2, w3, b3):
    rows = x.shape[0] * _CHUNK
    grid = (rows // _TILE,)
    full = lambda shape: pl.BlockSpec(shape, lambda i: (0, 0))
    return pl.pallas_call(
        _mlp_body,
        grid=grid,
        in_specs=[
            pl.BlockSpec((_TILE // _CHUNK, _CHUNK, 2 * DIM),
                         lambda i: (i, 0, 0)),
            full(w0.shape), full(b0.shape),
            full(w1.shape), full(b1.shape),
            full(w2.shape), full(b2.shape),
            full(w3.shape), full(b3.shape),
        ],
        out_specs=pl.BlockSpec((_TILE, 1), lambda i: (i, 0)),
        out_shape=jax.ShapeDtypeStruct((rows, 1), jnp.float32),
    )(x, w0, b0, w1, b1, w2, b2, w3, b3)


def kernel(item_index, user_index, item_emb, user_emb,
           W0, b0, W1, b1, W2, b2, W3, b3):
    uidx = user_index.astype(jnp.int32).reshape(BATCH // _CHUNK, _CHUNK)
    iidx = item_index.astype(jnp.int32).reshape(BATCH // _CHUNK, _CHUNK)

    wargs = (W0, b0.reshape(1, -1),
             W1, b1.reshape(1, -1),
             W2, b2.reshape(1, -1),
             W3.T, b3.reshape(1, -1))

    xs = []
    row0 = 0
    for srows in _SPLITS:
        xs.append(_sc_gather(row0, srows, uidx, iidx, user_emb, item_emb))
        row0 += srows
    outs = [_mlp(x, *wargs) for x in xs]
    if len(outs) == 1:
        return outs[0]
    return jnp.concatenate(outs, axis=0)


# speedup vs baseline: 1.1255x; 1.1241x over previous
import jax
import jax.numpy as jnp
from jax import lax
from jax.experimental import pallas as pl
from jax.experimental.pallas import tpu as pltpu
from jax.experimental.pallas import tpu_sc as plsc

BATCH = 16384
DIM = 128

_NC = 2
_NS = 16
_NW = _NC * _NS
_CHUNK = 128
_SPLITS = (8192, 8192)


def _make_sc_gather_body(row0, srows):
    _NCHUNK = srows // (_NW * _CHUNK)

    def body(uidx_hbm, iidx_hbm, user_emb_hbm, item_emb_hbm,
             x_hbm,
             uidx_v, iidx_v,
             ur0, ur1, ir0, ir1,
             g0, g1, s0, s1):
        wid = lax.axis_index("s") * _NC + lax.axis_index("c")
        idx_row0 = row0 // _CHUNK + wid * _NCHUNK
        ubuf, ibuf = (ur0, ur1), (ir0, ir1)
        gsem, ssem = (g0, g1), (s0, s1)
        stage = []
        for c in range(_NCHUNK):
            stage.append(pltpu.async_copy(uidx_hbm.at[pl.ds(idx_row0 + c, 1)],
                                          uidx_v.at[pl.ds(c, 1)], g0))
            stage.append(pltpu.async_copy(iidx_hbm.at[pl.ds(idx_row0 + c, 1)],
                                          iidx_v.at[pl.ds(c, 1)], g1))
        for cp in stage:
            cp.wait()

        def fire_gather(c, s):
            cu = pltpu.async_copy(user_emb_hbm.at[uidx_v.at[c]], ubuf[s], gsem[s])
            ci = pltpu.async_copy(item_emb_hbm.at[iidx_v.at[c]], ibuf[s], gsem[s])
            return cu, ci

        scat = [None, None]
        gath = [None, None]
        gath[0] = fire_gather(0, 0)
        for c in range(_NCHUNK):
            s = c % 2
            if c + 1 < _NCHUNK:
                if scat[1 - s] is not None:
                    scat[1 - s][0].wait()
                    scat[1 - s][1].wait()
                gath[1 - s] = fire_gather(c + 1, 1 - s)
            gath[s][0].wait()
            gath[s][1].wait()
            blk = wid * _NCHUNK + c
            su = pltpu.async_copy(
                ubuf[s], x_hbm.at[blk, slice(None), pl.ds(0, DIM)], ssem[s])
            si = pltpu.async_copy(
                ibuf[s], x_hbm.at[blk, slice(None), pl.ds(DIM, DIM)], ssem[s])
            scat[s] = (su, si)
        for s in range(2):
            if scat[s] is not None:
                scat[s][0].wait()
                scat[s][1].wait()

    return body


def _sc_gather(row0, srows, uidx, iidx, user_emb, item_emb):
    nchunk = srows // (_NW * _CHUNK)
    mesh = plsc.VectorSubcoreMesh(core_axis_name="c", subcore_axis_name="s")
    return pl.kernel(
        _make_sc_gather_body(row0, srows),
        out_type=jax.ShapeDtypeStruct((srows // _CHUNK, _CHUNK, 2 * DIM),
                                      jnp.float32),
        mesh=mesh,
        compiler_params=pltpu.CompilerParams(use_tc_tiling_on_sc=True),
        scratch_types=[
            pltpu.VMEM((nchunk, _CHUNK), jnp.int32),
            pltpu.VMEM((nchunk, _CHUNK), jnp.int32),
            pltpu.VMEM((_CHUNK, DIM), jnp.float32),
            pltpu.VMEM((_CHUNK, DIM), jnp.float32),
            pltpu.VMEM((_CHUNK, DIM), jnp.float32),
            pltpu.VMEM((_CHUNK, DIM), jnp.float32),
            pltpu.SemaphoreType.DMA,
            pltpu.SemaphoreType.DMA,
            pltpu.SemaphoreType.DMA,
            pltpu.SemaphoreType.DMA,
        ],
    )(uidx, iidx, user_emb, item_emb)


_TILE = 4096


def _dot_t(x, w):
    return lax.dot_general(x, w, (((1,), (1,)), ((), ())),
                           preferred_element_type=jnp.float32)


def _mlp_body(x_ref, w0_ref, b0_ref, w1_ref, b1_ref,
              w2_ref, b2_ref, w3_ref, b3_ref, out_ref):
    x = x_ref[...].reshape(_TILE, 2 * DIM)
    h = jnp.maximum(_dot_t(x, w0_ref[...]) + b0_ref[...], 0.0)
    h = jnp.maximum(_dot_t(h, w1_ref[...]) + b1_ref[...], 0.0)
    h = jnp.maximum(_dot_t(h, w2_ref[...]) + b2_ref[...], 0.0)
    h = jnp.dot(h, w3_ref[...], preferred_element_type=jnp.float32)
    h = jnp.maximum(h + b3_ref[...], 0.0)
    out_ref[...] = 1.0 / (1.0 + jnp.exp(-h))


def _mlp(x, w0, b0, w1, b1, w2, b2, w3, b3):
    rows = x.shape[0] * _CHUNK
    grid = (rows // _TILE,)
    full = lambda shape: pl.BlockSpec(shape, lambda i: (0, 0))
    return pl.pallas_call(
        _mlp_body,
        grid=grid,
        in_specs=[
            pl.BlockSpec((_TILE // _CHUNK, _CHUNK, 2 * DIM),
                         lambda i: (i, 0, 0)),
            full(w0.shape), full(b0.shape),
            full(w1.shape), full(b1.shape),
            full(w2.shape), full(b2.shape),
            full(w3.shape), full(b3.shape),
        ],
        out_specs=pl.BlockSpec((_TILE, 1), lambda i: (i, 0)),
        out_shape=jax.ShapeDtypeStruct((rows, 1), jnp.float32),
    )(x, w0, b0, w1, b1, w2, b2, w3, b3)


def kernel(item_index, user_index, item_emb, user_emb,
           W0, b0, W1, b1, W2, b2, W3, b3):
    uidx = user_index.astype(jnp.int32).reshape(BATCH // _CHUNK, _CHUNK)
    iidx = item_index.astype(jnp.int32).reshape(BATCH // _CHUNK, _CHUNK)

    wargs = (W0, b0.reshape(1, -1),
             W1, b1.reshape(1, -1),
             W2, b2.reshape(1, -1),
             W3.T, b3.reshape(1, -1))

    xs = []
    row0 = 0
    for srows in _SPLITS:
        xs.append(_sc_gather(row0, srows, uidx, iidx, user_emb, item_emb))
        row0 += srows
    outs = [_mlp(x, *wargs) for x in xs]
    if len(outs) == 1:
        return outs[0]
    return jnp.concatenate(outs, axis=0)
